# Initial kernel scaffold; baseline (speedup 1.0000x reference)
import jax
import jax.numpy as jnp
from jax.experimental import pallas as pl


def kernel(x, edges_weight, bn_gamma, bn_beta, gcn_W, gcn_b, qkv_w, qkv_b, out_w, out_b, p1_w, p1_b, p2_w, p2_b, edges_index):
    raise NotImplementedError("write your pallas kernel here")


# speedup vs baseline: 6.0189x; 6.0189x over previous
import functools

import jax
import jax.numpy as jnp
from jax import lax
from jax.experimental import pallas as pl
from jax.experimental.pallas import tpu as pltpu
from jax.experimental.pallas import tpu_sc as plsc

N = 10000
E = 320000
R = 2
D = 128
H = 8
NL = 3
H1 = 128
H2 = 64

NP = 10240
NT = 16
EPT = E // NT
ROWS_PT = NP // NT
K = 80
KD = 80
KN = 400

_mesh = plsc.VectorSubcoreMesh(core_axis_name="c", subcore_axis_name="s")


def _rsqrt16(d):
    i = lax.bitcast_convert_type(d, jnp.int32)
    y = lax.bitcast_convert_type(jnp.int32(0x5F3759DF) - (i >> 1), jnp.float32)
    for _ in range(3):
        y = y * (1.5 - 0.5 * d * y * y)
    return jnp.where(d > 0.0, y, 0.0)


@functools.partial(
    pl.kernel,
    out_type=jax.ShapeDtypeStruct((R * E,), jnp.float32),
    mesh=_mesh,
    compiler_params=pltpu.CompilerParams(needs_layout_passes=False),
    scratch_types=[
        pltpu.VMEM_SHARED((NP,), jnp.float32),
        pltpu.VMEM((ROWS_PT,), jnp.float32),
        pltpu.VMEM((KD,), jnp.int32),
        pltpu.VMEM((KD,), jnp.float32),
        pltpu.VMEM((NP,), jnp.float32),
        pltpu.VMEM((KN,), jnp.int32),
        pltpu.VMEM((KN,), jnp.int32),
        pltpu.VMEM((KN,), jnp.float32),
        pltpu.VMEM((KN,), jnp.float32),
    ],
)
def _sc_prep(edges, ew, norm_out, deg_sh, zv, dbuf, wbuf, disv, sb2, db2, wb2, nb2):
    r = lax.axis_index("c")
    s = lax.axis_index("s")
    src_base = r * (2 * E)
    dst_base = src_base + E
    w_base = r * E

    def z0(i, c):
        zv[pl.ds(i * 16, 16)] = jnp.zeros((16,), jnp.float32)
        return c
    lax.fori_loop(0, ROWS_PT // 16, z0, 0)
    pltpu.sync_copy(zv, deg_sh.at[pl.ds(pl.multiple_of(s * ROWS_PT, 8), ROWS_PT)])
    plsc.subcore_barrier()

    def body1(i, c):
        e0 = s * EPT + i * KD
        pltpu.sync_copy(edges.at[pl.ds(pl.multiple_of(dst_base + e0, 8), KD)], dbuf)
        pltpu.sync_copy(ew.at[pl.ds(pl.multiple_of(w_base + e0, 8), KD)], wbuf)
        pltpu.sync_copy(wbuf, deg_sh.at[dbuf], add=True)
        return c
    lax.fori_loop(0, EPT // KD, body1, 0)
    plsc.subcore_barrier()

    pltpu.sync_copy(deg_sh, disv)
    def body2(i, c):
        sl = pl.ds(i * 16, 16)
        disv[sl] = _rsqrt16(disv[sl])
        return c
    lax.fori_loop(0, NP // 16, body2, 0)

    def body3(i, c):
        e0 = s * EPT + i * KN
        pltpu.sync_copy(edges.at[pl.ds(pl.multiple_of(src_base + e0, 8), KN)], sb2)
        pltpu.sync_copy(edges.at[pl.ds(pl.multiple_of(dst_base + e0, 8), KN)], db2)
        pltpu.sync_copy(ew.at[pl.ds(pl.multiple_of(w_base + e0, 8), KN)], wb2)
        def inner(j, c2):
            sl = pl.ds(j * 16, 16)
            a = plsc.load_gather(disv, [sb2[sl]])
            b = plsc.load_gather(disv, [db2[sl]])
            nb2[sl] = a * b * wb2[sl]
            return c2
        lax.fori_loop(0, KN // 16, inner, 0)
        pltpu.sync_copy(nb2, norm_out.at[pl.ds(pl.multiple_of(w_base + e0, 8), KN)])
        return c
    lax.fori_loop(0, EPT // KN, body3, 0)


@functools.partial(
    pl.kernel,
    out_type=jax.ShapeDtypeStruct((R, N, D), jnp.float32),
    mesh=_mesh,
    compiler_params=pltpu.CompilerParams(needs_layout_passes=False),
    scratch_types=[
        pltpu.VMEM_SHARED((NP, D), jnp.float32),
        pltpu.VMEM((K, D), jnp.float32),
        pltpu.VMEM((K,), jnp.int32),
        pltpu.VMEM((K,), jnp.int32),
        pltpu.VMEM((K,), jnp.float32),
        pltpu.SemaphoreType.DMA,
    ],
)
def _sc_spmm(hw, edges, norm, agg, acc, rows, sbuf, dbuf, nbuf, sem):
    r = lax.axis_index("c")
    s = lax.axis_index("s")
    rbase = r * N
    src_base = r * (2 * E)
    dst_base = src_base + E
    w_base = r * E

    def zb(i, c):
        for v in range(8):
            rows[i, pl.ds(v * 16, 16)] = jnp.zeros((16,), jnp.float32)
        return c
    lax.fori_loop(0, K, zb, 0)
    row0 = pl.multiple_of(s * ROWS_PT, 8)
    def zc(i, c):
        pltpu.sync_copy(rows, acc.at[pl.ds(row0 + i * K, K)])
        return c
    lax.fori_loop(0, ROWS_PT // K, zc, 0)
    plsc.subcore_barrier()

    def body(i, c):
        e0 = s * EPT + i * K
        pltpu.sync_copy(edges.at[pl.ds(pl.multiple_of(src_base + e0, 8), K)], sbuf)
        pltpu.sync_copy(edges.at[pl.ds(pl.multiple_of(dst_base + e0, 8), K)], dbuf)
        pltpu.sync_copy(norm.at[pl.ds(pl.multiple_of(w_base + e0, 8), K)], nbuf)
        def off(j, c2):
            sl = pl.ds(j * 16, 16)
            sbuf[sl] = sbuf[sl] + rbase
            return c2
        lax.fori_loop(0, K // 16, off, 0)
        pltpu.async_copy(hw.at[sbuf], rows, sem).wait()
        def scale(j, c2):
            nb = plsc.load_gather(nbuf, [jnp.zeros((16,), jnp.int32) + j])
            for v in range(8):
                sl = pl.ds(v * 16, 16)
                rows[j, sl] = rows[j, sl] * nb
            return c2
        lax.fori_loop(0, K, scale, 0)
        pltpu.sync_copy(rows, acc.at[dbuf], add=True)
        return c
    lax.fori_loop(0, EPT // K, body, 0)
    plsc.subcore_barrier()

    @pl.when(s < NT - 1)
    def _():
        pltpu.sync_copy(acc.at[pl.ds(row0, ROWS_PT)], agg.at[r, pl.ds(row0, ROWS_PT)])
    @pl.when(s == NT - 1)
    def _():
        tail = N - (NT - 1) * ROWS_PT
        base = pl.multiple_of((NT - 1) * ROWS_PT, 8)
        pltpu.sync_copy(acc.at[pl.ds(base, tail)], agg.at[r, pl.ds(base, tail)])


def _dot_t(a, w):
    return lax.dot_general(a, w, (((1,), (1,)), ((), ())),
                           preferred_element_type=jnp.float32)


def _head_matrix():
    lane = lax.broadcasted_iota(jnp.int32, (D, H), 0)
    hh = lax.broadcasted_iota(jnp.int32, (D, H), 1)
    return (lane // (D // H) == hh).astype(jnp.float32)


def _tc_pre_body(x_ref, g_ref, b_ref, w0_ref, hw_ref):
    x = x_ref[...]
    s1 = jnp.sum(x, axis=0, keepdims=True)
    s2 = jnp.sum(x * x, axis=0, keepdims=True)
    mean = s1 / N
    var = s2 / N - mean * mean
    xn = (x - mean) * lax.rsqrt(var + 1e-5) * g_ref[...] + b_ref[...]
    hw0 = _dot_t(xn, w0_ref[...])
    hw_ref[0] = hw0
    hw_ref[1] = hw0


def _attn_core(agg_ref, gb_ref, qkvw_ref, qkvb_ref, ow_ref, ob_ref):
    gb = gb_ref[...]
    z0 = agg_ref[0] + gb
    z1 = agg_ref[1] + gb
    qkvw = qkvw_ref[...]
    qkvb = qkvb_ref[...]
    wq, wk, wv = qkvw[:D], qkvw[D:2 * D], qkvw[2 * D:]
    bq, bk, bv = qkvb[:, :D], qkvb[:, D:2 * D], qkvb[:, 2 * D:]
    q0 = _dot_t(z0, wq) + bq
    q1 = _dot_t(z1, wq) + bq
    k0 = _dot_t(z0, wk) + bk
    k1 = _dot_t(z1, wk) + bk
    v0 = _dot_t(z0, wv) + bv
    v1 = _dot_t(z1, wv) + bv

    M = _head_matrix()
    scale = 1.0 / jnp.sqrt(jnp.float32(D // H))
    def hsum(t):
        return lax.dot_general(t, M, (((1,), (0,)), ((), ())),
                               preferred_element_type=jnp.float32)
    s00 = hsum(q0 * k0) * scale
    s01 = hsum(q0 * k1) * scale
    s10 = hsum(q1 * k0) * scale
    s11 = hsum(q1 * k1) * scale

    m0 = jnp.maximum(s00, s01)
    e00 = jnp.exp(s00 - m0)
    e01 = jnp.exp(s01 - m0)
    a00 = e00 / (e00 + e01)
    a01 = e01 / (e00 + e01)
    m1 = jnp.maximum(s10, s11)
    e10 = jnp.exp(s10 - m1)
    e11 = jnp.exp(s11 - m1)
    a10 = e10 / (e10 + e11)
    a11 = e11 / (e10 + e11)

    def hexp(a):
        return lax.dot_general(a, M, (((1,), (1,)), ((), ())),
                               preferred_element_type=jnp.float32)
    o0 = hexp(a00) * v0 + hexp(a01) * v1
    o1 = hexp(a10) * v0 + hexp(a11) * v1
    ob = ob_ref[...]
    r0 = jnp.maximum(_dot_t(o0, ow_ref[...]) + ob, 0.0)
    r1 = jnp.maximum(_dot_t(o1, ow_ref[...]) + ob, 0.0)
    return r0, r1


def _tc_attn_body(agg_ref, gb_ref, qkvw_ref, qkvb_ref, ow_ref, ob_ref, wn_ref, out_ref):
    r0, r1 = _attn_core(agg_ref, gb_ref, qkvw_ref, qkvb_ref, ow_ref, ob_ref)
    wn = wn_ref[...]
    out_ref[0] = _dot_t(r0, wn)
    out_ref[1] = _dot_t(r1, wn)


def _tc_final_body(agg_ref, gb_ref, qkvw_ref, qkvb_ref, ow_ref, ob_ref,
                   p1w_ref, p1b_ref, p2w_ref, p2b_ref, out_ref):
    r0, r1 = _attn_core(agg_ref, gb_ref, qkvw_ref, qkvb_ref, ow_ref, ob_ref)
    p1w, p1b = p1w_ref[...], p1b_ref[...]
    p2w, p2b = p2w_ref[...], p2b_ref[...]
    def mlp(t):
        h = _dot_t(t, p1w) + p1b
        h = jnp.where(h > 0, h, 0.01 * h)
        h = _dot_t(h, p2w) + p2b
        return jnp.where(h > 0, h, 0.01 * h)
    out_ref[0] = mlp(r0)
    out_ref[1] = mlp(r1)


_BN = 1000


def _full(shape):
    return pl.BlockSpec(shape, lambda i: tuple(0 for _ in shape))


def _tc_pre(x, g, b, w0):
    return pl.pallas_call(
        _tc_pre_body,
        out_shape=jax.ShapeDtypeStruct((R, N, D), jnp.float32),
    )(x, g, b, w0)


def _tc_attn(agg, gb, qkvw, qkvb, ow, ob, wn):
    grid = (N // _BN,)
    return pl.pallas_call(
        _tc_attn_body,
        grid=grid,
        in_specs=[
            pl.BlockSpec((R, _BN, D), lambda i: (0, i, 0)),
            _full((1, D)), _full((3 * D, D)), _full((1, 3 * D)),
            _full((D, D)), _full((1, D)), _full((D, D)),
        ],
        out_specs=pl.BlockSpec((R, _BN, D), lambda i: (0, i, 0)),
        out_shape=jax.ShapeDtypeStruct((R, N, D), jnp.float32),
    )(agg, gb, qkvw, qkvb, ow, ob, wn)


def _tc_final(agg, gb, qkvw, qkvb, ow, ob, p1w, p1b, p2w, p2b):
    grid = (N // _BN,)
    return pl.pallas_call(
        _tc_final_body,
        grid=grid,
        in_specs=[
            pl.BlockSpec((R, _BN, D), lambda i: (0, i, 0)),
            _full((1, D)), _full((3 * D, D)), _full((1, 3 * D)),
            _full((D, D)), _full((1, D)),
            _full((H1, D)), _full((1, H1)), _full((H2, H1)), _full((1, H2)),
        ],
        out_specs=pl.BlockSpec((R, _BN, H2), lambda i: (0, i, 0)),
        out_shape=jax.ShapeDtypeStruct((R, N, H2), jnp.float32),
    )(agg, gb, qkvw, qkvb, ow, ob, p1w, p1b, p2w, p2b)


def kernel(x, edges_weight, bn_gamma, bn_beta, gcn_W, gcn_b, qkv_w, qkv_b,
           out_w, out_b, p1_w, p1_b, p2_w, p2_b, edges_index):
    g = bn_gamma.reshape(1, D)
    b = bn_beta.reshape(1, D)
    hw = _tc_pre(x, g, b, gcn_W[0])
    edges_flat = edges_index.reshape(R * 2 * E)
    ew_flat = edges_weight.reshape(R * E)
    norm = _sc_prep(edges_flat, ew_flat)
    out = None
    for i in range(NL):
        agg = _sc_spmm(hw.reshape(R * N, D), edges_flat, norm)
        gb = gcn_b[i].reshape(1, D)
        qb = qkv_b[i].reshape(1, 3 * D)
        ob = out_b[i].reshape(1, D)
        if i < NL - 1:
            hw = _tc_attn(agg, gb, qkv_w[i], qb, out_w[i], ob, gcn_W[i + 1])
        else:
            out = _tc_final(agg, gb, qkv_w[i], qb, out_w[i], ob,
                            p1_w, p1_b.reshape(1, H1), p2_w, p2_b.reshape(1, H2))
    return out


# speedup vs baseline: 10.5424x; 1.7516x over previous
import functools

import jax
import jax.numpy as jnp
from jax import lax
from jax.experimental import pallas as pl
from jax.experimental.pallas import tpu as pltpu
from jax.experimental.pallas import tpu_sc as plsc

N = 10000
E = 320000
R = 2
D = 128
H = 8
NL = 3
H1 = 128
H2 = 64

NP = 10240
NT = 16
EPT = E // NT
ROWS_PT = NP // NT
K = 80
KD = 80
KN = 400

_mesh = plsc.VectorSubcoreMesh(core_axis_name="c", subcore_axis_name="s")


def _rsqrt16(d):
    i = lax.bitcast_convert_type(d, jnp.int32)
    y = lax.bitcast_convert_type(jnp.int32(0x5F3759DF) - (i >> 1), jnp.float32)
    for _ in range(3):
        y = y * (1.5 - 0.5 * d * y * y)
    return jnp.where(d > 0.0, y, 0.0)


@functools.partial(
    pl.kernel,
    out_type=jax.ShapeDtypeStruct((R * E,), jnp.float32),
    mesh=_mesh,
    compiler_params=pltpu.CompilerParams(needs_layout_passes=False),
    scratch_types=[
        pltpu.VMEM_SHARED((NP,), jnp.float32),
        pltpu.VMEM((ROWS_PT,), jnp.float32),
        pltpu.VMEM((KD,), jnp.int32),
        pltpu.VMEM((KD,), jnp.float32),
        pltpu.VMEM((NP,), jnp.float32),
        pltpu.VMEM((KN,), jnp.int32),
        pltpu.VMEM((KN,), jnp.int32),
        pltpu.VMEM((KN,), jnp.float32),
        pltpu.VMEM((KN,), jnp.float32),
    ],
)
def _sc_prep(edges, ew, norm_out, deg_sh, zv, dbuf, wbuf, disv, sb2, db2, wb2, nb2):
    r = lax.axis_index("c")
    s = lax.axis_index("s")
    src_base = r * (2 * E)
    dst_base = src_base + E
    w_base = r * E

    def z0(i, c):
        zv[pl.ds(i * 16, 16)] = jnp.zeros((16,), jnp.float32)
        return c
    lax.fori_loop(0, ROWS_PT // 16, z0, 0)
    pltpu.sync_copy(zv, deg_sh.at[pl.ds(pl.multiple_of(s * ROWS_PT, 8), ROWS_PT)])
    plsc.subcore_barrier()

    def body1(i, c):
        e0 = s * EPT + i * KD
        pltpu.sync_copy(edges.at[pl.ds(pl.multiple_of(dst_base + e0, 8), KD)], dbuf)
        pltpu.sync_copy(ew.at[pl.ds(pl.multiple_of(w_base + e0, 8), KD)], wbuf)
        pltpu.sync_copy(wbuf, deg_sh.at[dbuf], add=True)
        return c
    lax.fori_loop(0, EPT // KD, body1, 0)
    plsc.subcore_barrier()

    pltpu.sync_copy(deg_sh, disv)
    def body2(i, c):
        sl = pl.ds(i * 16, 16)
        disv[sl] = _rsqrt16(disv[sl])
        return c
    lax.fori_loop(0, NP // 16, body2, 0)

    def body3(i, c):
        e0 = s * EPT + i * KN
        pltpu.sync_copy(edges.at[pl.ds(pl.multiple_of(src_base + e0, 8), KN)], sb2)
        pltpu.sync_copy(edges.at[pl.ds(pl.multiple_of(dst_base + e0, 8), KN)], db2)
        pltpu.sync_copy(ew.at[pl.ds(pl.multiple_of(w_base + e0, 8), KN)], wb2)
        def inner(j, c2):
            sl = pl.ds(j * 16, 16)
            a = plsc.load_gather(disv, [sb2[sl]])
            b = plsc.load_gather(disv, [db2[sl]])
            nb2[sl] = a * b * wb2[sl]
            return c2
        lax.fori_loop(0, KN // 16, inner, 0)
        pltpu.sync_copy(nb2, norm_out.at[pl.ds(pl.multiple_of(w_base + e0, 8), KN)])
        return c
    lax.fori_loop(0, EPT // KN, body3, 0)


KS = 32
NC = EPT // KS
NB = 5
NM = 25


@functools.partial(
    pl.kernel,
    out_type=jax.ShapeDtypeStruct((R, N, D), jnp.float32),
    mesh=_mesh,
    compiler_params=pltpu.CompilerParams(needs_layout_passes=False),
    scratch_types=[
        pltpu.VMEM_SHARED((NP, D), jnp.float32),
        [pltpu.VMEM((KS, D), jnp.float32)] * NB,
        [pltpu.VMEM((KS,), jnp.int32)] * NM,
        [pltpu.VMEM((KS,), jnp.int32)] * NM,
        [pltpu.VMEM((KS,), jnp.float32)] * NM,
        [pltpu.SemaphoreType.DMA] * NB,
        [pltpu.SemaphoreType.DMA] * NM,
    ],
)
def _sc_spmm(hw, edges, norm, agg, acc, rows, six, dix, nbf,
             sem_g, sem_m):
    r = lax.axis_index("c")
    s = lax.axis_index("s")
    rbase = r * N
    src_base = r * (2 * E) + s * EPT
    dst_base = src_base + E
    w_base = r * E + s * EPT

    def zb(i, c):
        for v in range(8):
            rows[0][i, pl.ds(v * 16, 16)] = jnp.zeros((16,), jnp.float32)
        return c
    lax.fori_loop(0, KS, zb, 0)
    row0 = pl.multiple_of(s * ROWS_PT, 8)
    def zc(i, c):
        pltpu.sync_copy(rows[0], acc.at[pl.ds(row0 + i * KS, KS)])
        return c
    lax.fori_loop(0, ROWS_PT // KS, zc, 0)
    plsc.subcore_barrier()

    def meta(i, m):
        e0 = i * KS
        pltpu.async_copy(edges.at[pl.ds(pl.multiple_of(src_base + e0, 8), KS)],
                         six[m], sem_m[m])
        pltpu.async_copy(edges.at[pl.ds(pl.multiple_of(dst_base + e0, 8), KS)],
                         dix[m], sem_m[m])
        pltpu.async_copy(norm.at[pl.ds(pl.multiple_of(w_base + e0, 8), KS)],
                         nbf[m], sem_m[m])

    def wait_m(m):
        pltpu.make_async_copy(edges.at[pl.ds(0, KS)], six[m], sem_m[m]).wait()
        pltpu.make_async_copy(edges.at[pl.ds(0, KS)], dix[m], sem_m[m]).wait()
        pltpu.make_async_copy(norm.at[pl.ds(0, KS)], nbf[m], sem_m[m]).wait()

    def offset_src(m):
        for v in range(KS // 16):
            sl = pl.ds(v * 16, 16)
            six[m][sl] = six[m][sl] + rbase

    def wait_g(b):
        pltpu.make_async_copy(hw.at[pl.ds(0, KS)], rows[b], sem_g[b]).wait()

    def scale_scatter(t):
        b = t % NB
        wait_g(b)
        def scale(j, c2):
            nb = plsc.load_gather(nbf[t], [jnp.zeros((16,), jnp.int32) + j])
            for v in range(8):
                sl = pl.ds(v * 16, 16)
                rows[b][j, sl] = rows[b][j, sl] * nb
            return c2
        lax.fori_loop(0, KS, scale, 0)
        pltpu.sync_copy(rows[b], acc.at[dix[t]], add=True)

    def next_gather(g, t):
        t1 = (t + 1) % NM
        wait_m(t1)
        offset_src(t1)
        pltpu.async_copy(hw.at[six[t1]], rows[(t + 1) % NB], sem_g[(t + 1) % NB])

    for t in range(NM):
        meta(t, t)
    wait_m(0)
    offset_src(0)
    pltpu.async_copy(hw.at[six[0]], rows[0], sem_g[0])

    def outer(g, c):
        for t in range(NM):
            next_gather(g, t)
            scale_scatter(t)
            meta((g + 1) * NM + t, t)
        return c
    lax.fori_loop(0, NC // NM - 1, outer, 0)
    gl = NC // NM - 1
    for t in range(NM):
        if t < NM - 1:
            next_gather(gl, t)
        scale_scatter(t)
    plsc.subcore_barrier()

    @pl.when(s < NT - 1)
    def _():
        pltpu.sync_copy(acc.at[pl.ds(row0, ROWS_PT)], agg.at[r, pl.ds(row0, ROWS_PT)])
    @pl.when(s == NT - 1)
    def _():
        tail = N - (NT - 1) * ROWS_PT
        base = pl.multiple_of((NT - 1) * ROWS_PT, 8)
        pltpu.sync_copy(acc.at[pl.ds(base, tail)], agg.at[r, pl.ds(base, tail)])


def _dot_t(a, w):
    return lax.dot_general(a, w, (((1,), (1,)), ((), ())),
                           preferred_element_type=jnp.float32)


def _head_matrix():
    lane = lax.broadcasted_iota(jnp.int32, (D, H), 0)
    hh = lax.broadcasted_iota(jnp.int32, (D, H), 1)
    return (lane // (D // H) == hh).astype(jnp.float32)


def _tc_pre_body(x_ref, g_ref, b_ref, w0_ref, hw_ref):
    x = x_ref[...]
    s1 = jnp.sum(x, axis=0, keepdims=True)
    s2 = jnp.sum(x * x, axis=0, keepdims=True)
    mean = s1 / N
    var = s2 / N - mean * mean
    xn = (x - mean) * lax.rsqrt(var + 1e-5) * g_ref[...] + b_ref[...]
    hw0 = _dot_t(xn, w0_ref[...])
    hw_ref[0] = hw0
    hw_ref[1] = hw0


def _attn_core(agg_ref, gb_ref, qkvw_ref, qkvb_ref, ow_ref, ob_ref):
    gb = gb_ref[...]
    z0 = agg_ref[0] + gb
    z1 = agg_ref[1] + gb
    qkvw = qkvw_ref[...]
    qkvb = qkvb_ref[...]
    wq, wk, wv = qkvw[:D], qkvw[D:2 * D], qkvw[2 * D:]
    bq, bk, bv = qkvb[:, :D], qkvb[:, D:2 * D], qkvb[:, 2 * D:]
    q0 = _dot_t(z0, wq) + bq
    q1 = _dot_t(z1, wq) + bq
    k0 = _dot_t(z0, wk) + bk
    k1 = _dot_t(z1, wk) + bk
    v0 = _dot_t(z0, wv) + bv
    v1 = _dot_t(z1, wv) + bv

    M = _head_matrix()
    scale = 1.0 / jnp.sqrt(jnp.float32(D // H))
    def hsum(t):
        return lax.dot_general(t, M, (((1,), (0,)), ((), ())),
                               preferred_element_type=jnp.float32)
    s00 = hsum(q0 * k0) * scale
    s01 = hsum(q0 * k1) * scale
    s10 = hsum(q1 * k0) * scale
    s11 = hsum(q1 * k1) * scale

    m0 = jnp.maximum(s00, s01)
    e00 = jnp.exp(s00 - m0)
    e01 = jnp.exp(s01 - m0)
    a00 = e00 / (e00 + e01)
    a01 = e01 / (e00 + e01)
    m1 = jnp.maximum(s10, s11)
    e10 = jnp.exp(s10 - m1)
    e11 = jnp.exp(s11 - m1)
    a10 = e10 / (e10 + e11)
    a11 = e11 / (e10 + e11)

    def hexp(a):
        return lax.dot_general(a, M, (((1,), (1,)), ((), ())),
                               preferred_element_type=jnp.float32)
    o0 = hexp(a00) * v0 + hexp(a01) * v1
    o1 = hexp(a10) * v0 + hexp(a11) * v1
    ob = ob_ref[...]
    r0 = jnp.maximum(_dot_t(o0, ow_ref[...]) + ob, 0.0)
    r1 = jnp.maximum(_dot_t(o1, ow_ref[...]) + ob, 0.0)
    return r0, r1


def _tc_attn_body(agg_ref, gb_ref, qkvw_ref, qkvb_ref, ow_ref, ob_ref, wn_ref, out_ref):
    r0, r1 = _attn_core(agg_ref, gb_ref, qkvw_ref, qkvb_ref, ow_ref, ob_ref)
    wn = wn_ref[...]
    out_ref[0] = _dot_t(r0, wn)
    out_ref[1] = _dot_t(r1, wn)


def _tc_final_body(agg_ref, gb_ref, qkvw_ref, qkvb_ref, ow_ref, ob_ref,
                   p1w_ref, p1b_ref, p2w_ref, p2b_ref, out_ref):
    r0, r1 = _attn_core(agg_ref, gb_ref, qkvw_ref, qkvb_ref, ow_ref, ob_ref)
    p1w, p1b = p1w_ref[...], p1b_ref[...]
    p2w, p2b = p2w_ref[...], p2b_ref[...]
    def mlp(t):
        h = _dot_t(t, p1w) + p1b
        h = jnp.where(h > 0, h, 0.01 * h)
        h = _dot_t(h, p2w) + p2b
        return jnp.where(h > 0, h, 0.01 * h)
    out_ref[0] = mlp(r0)
    out_ref[1] = mlp(r1)


_BN = 1000


def _full(shape):
    return pl.BlockSpec(shape, lambda i: tuple(0 for _ in shape))


def _tc_pre(x, g, b, w0):
    return pl.pallas_call(
        _tc_pre_body,
        out_shape=jax.ShapeDtypeStruct((R, N, D), jnp.float32),
    )(x, g, b, w0)


def _tc_attn(agg, gb, qkvw, qkvb, ow, ob, wn):
    grid = (N // _BN,)
    return pl.pallas_call(
        _tc_attn_body,
        grid=grid,
        in_specs=[
            pl.BlockSpec((R, _BN, D), lambda i: (0, i, 0)),
            _full((1, D)), _full((3 * D, D)), _full((1, 3 * D)),
            _full((D, D)), _full((1, D)), _full((D, D)),
        ],
        out_specs=pl.BlockSpec((R, _BN, D), lambda i: (0, i, 0)),
        out_shape=jax.ShapeDtypeStruct((R, N, D), jnp.float32),
    )(agg, gb, qkvw, qkvb, ow, ob, wn)


def _tc_final(agg, gb, qkvw, qkvb, ow, ob, p1w, p1b, p2w, p2b):
    grid = (N // _BN,)
    return pl.pallas_call(
        _tc_final_body,
        grid=grid,
        in_specs=[
            pl.BlockSpec((R, _BN, D), lambda i: (0, i, 0)),
            _full((1, D)), _full((3 * D, D)), _full((1, 3 * D)),
            _full((D, D)), _full((1, D)),
            _full((H1, D)), _full((1, H1)), _full((H2, H1)), _full((1, H2)),
        ],
        out_specs=pl.BlockSpec((R, _BN, H2), lambda i: (0, i, 0)),
        out_shape=jax.ShapeDtypeStruct((R, N, H2), jnp.float32),
    )(agg, gb, qkvw, qkvb, ow, ob, p1w, p1b, p2w, p2b)


def kernel(x, edges_weight, bn_gamma, bn_beta, gcn_W, gcn_b, qkv_w, qkv_b,
           out_w, out_b, p1_w, p1_b, p2_w, p2_b, edges_index):
    g = bn_gamma.reshape(1, D)
    b = bn_beta.reshape(1, D)
    hw = _tc_pre(x, g, b, gcn_W[0])
    edges_flat = edges_index.reshape(R * 2 * E)
    ew_flat = edges_weight.reshape(R * E)
    norm = _sc_prep(edges_flat, ew_flat)
    out = None
    for i in range(NL):
        agg = _sc_spmm(hw.reshape(R * N, D), edges_flat, norm)
        gb = gcn_b[i].reshape(1, D)
        qb = qkv_b[i].reshape(1, 3 * D)
        ob = out_b[i].reshape(1, D)
        if i < NL - 1:
            hw = _tc_attn(agg, gb, qkv_w[i], qb, out_w[i], ob, gcn_W[i + 1])
        else:
            out = _tc_final(agg, gb, qkv_w[i], qb, out_w[i], ob,
                            p1_w, p1_b.reshape(1, H1), p2_w, p2_b.reshape(1, H2))
    return out


# speedup vs baseline: 13.0695x; 1.2397x over previous
import functools

import jax
import jax.numpy as jnp
from jax import lax
from jax.experimental import pallas as pl
from jax.experimental.pallas import tpu as pltpu
from jax.experimental.pallas import tpu_sc as plsc

N = 10000
E = 320000
R = 2
D = 128
H = 8
NL = 3
H1 = 128
H2 = 64

NP = 10240
NT = 16
EPT = E // NT
ROWS_PT = NP // NT
K = 80
KD = 80
KN = 400

_mesh = plsc.VectorSubcoreMesh(core_axis_name="c", subcore_axis_name="s")


def _rsqrt16(d):
    i = lax.bitcast_convert_type(d, jnp.int32)
    y = lax.bitcast_convert_type(jnp.int32(0x5F3759DF) - (i >> 1), jnp.float32)
    for _ in range(3):
        y = y * (1.5 - 0.5 * d * y * y)
    return jnp.where(d > 0.0, y, 0.0)


@functools.partial(
    pl.kernel,
    out_type=jax.ShapeDtypeStruct((R * E,), jnp.float32),
    mesh=_mesh,
    compiler_params=pltpu.CompilerParams(needs_layout_passes=False),
    scratch_types=[
        pltpu.VMEM_SHARED((NP,), jnp.float32),
        pltpu.VMEM((ROWS_PT,), jnp.float32),
        pltpu.VMEM((KD,), jnp.int32),
        pltpu.VMEM((KD,), jnp.float32),
        pltpu.VMEM((NP,), jnp.float32),
        pltpu.VMEM((KN,), jnp.int32),
        pltpu.VMEM((KN,), jnp.int32),
        pltpu.VMEM((KN,), jnp.float32),
        pltpu.VMEM((KN,), jnp.float32),
    ],
)
def _sc_prep(edges, ew, norm_out, deg_sh, zv, dbuf, wbuf, disv, sb2, db2, wb2, nb2):
    r = lax.axis_index("c")
    s = lax.axis_index("s")
    src_base = r * (2 * E)
    dst_base = src_base + E
    w_base = r * E

    def z0(i, c):
        zv[pl.ds(i * 16, 16)] = jnp.zeros((16,), jnp.float32)
        return c
    lax.fori_loop(0, ROWS_PT // 16, z0, 0)
    pltpu.sync_copy(zv, deg_sh.at[pl.ds(pl.multiple_of(s * ROWS_PT, 8), ROWS_PT)])
    plsc.subcore_barrier()

    def body1(i, c):
        e0 = s * EPT + i * KD
        pltpu.sync_copy(edges.at[pl.ds(pl.multiple_of(dst_base + e0, 8), KD)], dbuf)
        pltpu.sync_copy(ew.at[pl.ds(pl.multiple_of(w_base + e0, 8), KD)], wbuf)
        pltpu.sync_copy(wbuf, deg_sh.at[dbuf], add=True)
        return c
    lax.fori_loop(0, EPT // KD, body1, 0)
    plsc.subcore_barrier()

    pltpu.sync_copy(deg_sh, disv)
    def body2(i, c):
        sl = pl.ds(i * 16, 16)
        disv[sl] = _rsqrt16(disv[sl])
        return c
    lax.fori_loop(0, NP // 16, body2, 0)

    def body3(i, c):
        e0 = s * EPT + i * KN
        pltpu.sync_copy(edges.at[pl.ds(pl.multiple_of(src_base + e0, 8), KN)], sb2)
        pltpu.sync_copy(edges.at[pl.ds(pl.multiple_of(dst_base + e0, 8), KN)], db2)
        pltpu.sync_copy(ew.at[pl.ds(pl.multiple_of(w_base + e0, 8), KN)], wb2)
        def inner(j, c2):
            sl = pl.ds(j * 16, 16)
            a = plsc.load_gather(disv, [sb2[sl]])
            b = plsc.load_gather(disv, [db2[sl]])
            nb2[sl] = a * b * wb2[sl]
            return c2
        lax.fori_loop(0, KN // 16, inner, 0)
        pltpu.sync_copy(nb2, norm_out.at[pl.ds(pl.multiple_of(w_base + e0, 8), KN)])
        return c
    lax.fori_loop(0, EPT // KN, body3, 0)


KS = 80
NC = EPT // KS
NB = 2
NM = 10


@functools.partial(
    pl.kernel,
    out_type=jax.ShapeDtypeStruct((R, N, D), jnp.float32),
    mesh=_mesh,
    compiler_params=pltpu.CompilerParams(needs_layout_passes=False),
    scratch_types=[
        pltpu.VMEM_SHARED((NP, D), jnp.float32),
        [pltpu.VMEM((KS, D), jnp.float32)] * NB,
        [pltpu.VMEM((KS,), jnp.int32)] * NM,
        [pltpu.VMEM((KS,), jnp.int32)] * NM,
        [pltpu.VMEM((KS,), jnp.float32)] * NM,
        [pltpu.SemaphoreType.DMA] * NB,
        [pltpu.SemaphoreType.DMA] * NM,
    ],
)
def _sc_spmm(hw, edges, norm, agg, acc, rows, six, dix, nbf,
             sem_g, sem_m):
    r = lax.axis_index("c")
    s = lax.axis_index("s")
    rbase = r * N
    src_base = r * (2 * E) + s * EPT
    dst_base = src_base + E
    w_base = r * E + s * EPT

    def zb(i, c):
        for v in range(8):
            rows[0][i, pl.ds(v * 16, 16)] = jnp.zeros((16,), jnp.float32)
        return c
    lax.fori_loop(0, KS, zb, 0)
    row0 = pl.multiple_of(s * ROWS_PT, 8)
    def zc(i, c):
        pltpu.sync_copy(rows[0], acc.at[pl.ds(row0 + i * KS, KS)])
        return c
    lax.fori_loop(0, ROWS_PT // KS, zc, 0)
    plsc.subcore_barrier()

    def meta(i, m):
        e0 = i * KS
        pltpu.async_copy(edges.at[pl.ds(pl.multiple_of(src_base + e0, 8), KS)],
                         six[m], sem_m[m])
        pltpu.async_copy(edges.at[pl.ds(pl.multiple_of(dst_base + e0, 8), KS)],
                         dix[m], sem_m[m])
        pltpu.async_copy(norm.at[pl.ds(pl.multiple_of(w_base + e0, 8), KS)],
                         nbf[m], sem_m[m])

    def wait_m(m):
        pltpu.make_async_copy(edges.at[pl.ds(0, KS)], six[m], sem_m[m]).wait()
        pltpu.make_async_copy(edges.at[pl.ds(0, KS)], dix[m], sem_m[m]).wait()
        pltpu.make_async_copy(norm.at[pl.ds(0, KS)], nbf[m], sem_m[m]).wait()

    def offset_src(m):
        for v in range(KS // 16):
            sl = pl.ds(v * 16, 16)
            six[m][sl] = six[m][sl] + rbase

    def wait_g(b):
        pltpu.make_async_copy(hw.at[pl.ds(0, KS)], rows[b], sem_g[b]).wait()

    def scale_scatter(t):
        b = t % NB
        wait_g(b)
        def scale(j, c2):
            nb = plsc.load_gather(nbf[t], [jnp.zeros((16,), jnp.int32) + j])
            for v in range(8):
                sl = pl.ds(v * 16, 16)
                rows[b][j, sl] = rows[b][j, sl] * nb
            return c2
        lax.fori_loop(0, KS, scale, 0)
        pltpu.sync_copy(rows[b], acc.at[dix[t]], add=True)

    def next_gather(g, t):
        t1 = (t + 1) % NM
        wait_m(t1)
        offset_src(t1)
        pltpu.async_copy(hw.at[six[t1]], rows[(t + 1) % NB], sem_g[(t + 1) % NB])

    for t in range(NM):
        meta(t, t)
    wait_m(0)
    offset_src(0)
    pltpu.async_copy(hw.at[six[0]], rows[0], sem_g[0])

    def outer(g, c):
        for t in range(NM):
            next_gather(g, t)
            scale_scatter(t)
            meta((g + 1) * NM + t, t)
        return c
    lax.fori_loop(0, NC // NM - 1, outer, 0)
    gl = NC // NM - 1
    for t in range(NM):
        if t < NM - 1:
            next_gather(gl, t)
        scale_scatter(t)
    plsc.subcore_barrier()

    @pl.when(s < NT - 1)
    def _():
        pltpu.sync_copy(acc.at[pl.ds(row0, ROWS_PT)], agg.at[r, pl.ds(row0, ROWS_PT)])
    @pl.when(s == NT - 1)
    def _():
        tail = N - (NT - 1) * ROWS_PT
        base = pl.multiple_of((NT - 1) * ROWS_PT, 8)
        pltpu.sync_copy(acc.at[pl.ds(base, tail)], agg.at[r, pl.ds(base, tail)])


def _dot_t(a, w):
    return lax.dot_general(a, w, (((1,), (1,)), ((), ())),
                           preferred_element_type=jnp.float32)


def _head_matrix():
    lane = lax.broadcasted_iota(jnp.int32, (D, H), 0)
    hh = lax.broadcasted_iota(jnp.int32, (D, H), 1)
    return (lane // (D // H) == hh).astype(jnp.float32)


def _tc_pre_body(x_ref, g_ref, b_ref, w0_ref, hw_ref):
    x = x_ref[...]
    s1 = jnp.sum(x, axis=0, keepdims=True)
    s2 = jnp.sum(x * x, axis=0, keepdims=True)
    mean = s1 / N
    var = s2 / N - mean * mean
    xn = (x - mean) * lax.rsqrt(var + 1e-5) * g_ref[...] + b_ref[...]
    hw0 = _dot_t(xn, w0_ref[...])
    hw_ref[0] = hw0
    hw_ref[1] = hw0


def _attn_core(agg_ref, gb_ref, qkvw_ref, qkvb_ref, ow_ref, ob_ref):
    gb = gb_ref[...]
    z0 = agg_ref[0] + gb
    z1 = agg_ref[1] + gb
    qkvw = qkvw_ref[...]
    qkvb = qkvb_ref[...]
    wq, wk, wv = qkvw[:D], qkvw[D:2 * D], qkvw[2 * D:]
    bq, bk, bv = qkvb[:, :D], qkvb[:, D:2 * D], qkvb[:, 2 * D:]
    q0 = _dot_t(z0, wq) + bq
    q1 = _dot_t(z1, wq) + bq
    k0 = _dot_t(z0, wk) + bk
    k1 = _dot_t(z1, wk) + bk
    v0 = _dot_t(z0, wv) + bv
    v1 = _dot_t(z1, wv) + bv

    M = _head_matrix()
    scale = 1.0 / jnp.sqrt(jnp.float32(D // H))
    def hsum(t):
        return lax.dot_general(t, M, (((1,), (0,)), ((), ())),
                               preferred_element_type=jnp.float32)
    s00 = hsum(q0 * k0) * scale
    s01 = hsum(q0 * k1) * scale
    s10 = hsum(q1 * k0) * scale
    s11 = hsum(q1 * k1) * scale

    m0 = jnp.maximum(s00, s01)
    e00 = jnp.exp(s00 - m0)
    e01 = jnp.exp(s01 - m0)
    a00 = e00 / (e00 + e01)
    a01 = e01 / (e00 + e01)
    m1 = jnp.maximum(s10, s11)
    e10 = jnp.exp(s10 - m1)
    e11 = jnp.exp(s11 - m1)
    a10 = e10 / (e10 + e11)
    a11 = e11 / (e10 + e11)

    def hexp(a):
        return lax.dot_general(a, M, (((1,), (1,)), ((), ())),
                               preferred_element_type=jnp.float32)
    o0 = hexp(a00) * v0 + hexp(a01) * v1
    o1 = hexp(a10) * v0 + hexp(a11) * v1
    ob = ob_ref[...]
    r0 = jnp.maximum(_dot_t(o0, ow_ref[...]) + ob, 0.0)
    r1 = jnp.maximum(_dot_t(o1, ow_ref[...]) + ob, 0.0)
    return r0, r1


def _tc_attn_body(agg_ref, gb_ref, qkvw_ref, qkvb_ref, ow_ref, ob_ref, wn_ref, out_ref):
    r0, r1 = _attn_core(agg_ref, gb_ref, qkvw_ref, qkvb_ref, ow_ref, ob_ref)
    wn = wn_ref[...]
    out_ref[0] = _dot_t(r0, wn)
    out_ref[1] = _dot_t(r1, wn)


def _tc_final_body(agg_ref, gb_ref, qkvw_ref, qkvb_ref, ow_ref, ob_ref,
                   p1w_ref, p1b_ref, p2w_ref, p2b_ref, out_ref):
    r0, r1 = _attn_core(agg_ref, gb_ref, qkvw_ref, qkvb_ref, ow_ref, ob_ref)
    p1w, p1b = p1w_ref[...], p1b_ref[...]
    p2w, p2b = p2w_ref[...], p2b_ref[...]
    def mlp(t):
        h = _dot_t(t, p1w) + p1b
        h = jnp.where(h > 0, h, 0.01 * h)
        h = _dot_t(h, p2w) + p2b
        return jnp.where(h > 0, h, 0.01 * h)
    out_ref[0] = mlp(r0)
    out_ref[1] = mlp(r1)


_BN = 1000


def _full(shape):
    return pl.BlockSpec(shape, lambda i: tuple(0 for _ in shape))


def _tc_pre(x, g, b, w0):
    return pl.pallas_call(
        _tc_pre_body,
        out_shape=jax.ShapeDtypeStruct((R, N, D), jnp.float32),
    )(x, g, b, w0)


def _tc_attn(agg, gb, qkvw, qkvb, ow, ob, wn):
    grid = (N // _BN,)
    return pl.pallas_call(
        _tc_attn_body,
        grid=grid,
        in_specs=[
            pl.BlockSpec((R, _BN, D), lambda i: (0, i, 0)),
            _full((1, D)), _full((3 * D, D)), _full((1, 3 * D)),
            _full((D, D)), _full((1, D)), _full((D, D)),
        ],
        out_specs=pl.BlockSpec((R, _BN, D), lambda i: (0, i, 0)),
        out_shape=jax.ShapeDtypeStruct((R, N, D), jnp.float32),
    )(agg, gb, qkvw, qkvb, ow, ob, wn)


def _tc_final(agg, gb, qkvw, qkvb, ow, ob, p1w, p1b, p2w, p2b):
    grid = (N // _BN,)
    return pl.pallas_call(
        _tc_final_body,
        grid=grid,
        in_specs=[
            pl.BlockSpec((R, _BN, D), lambda i: (0, i, 0)),
            _full((1, D)), _full((3 * D, D)), _full((1, 3 * D)),
            _full((D, D)), _full((1, D)),
            _full((H1, D)), _full((1, H1)), _full((H2, H1)), _full((1, H2)),
        ],
        out_specs=pl.BlockSpec((R, _BN, H2), lambda i: (0, i, 0)),
        out_shape=jax.ShapeDtypeStruct((R, N, H2), jnp.float32),
    )(agg, gb, qkvw, qkvb, ow, ob, p1w, p1b, p2w, p2b)


def kernel(x, edges_weight, bn_gamma, bn_beta, gcn_W, gcn_b, qkv_w, qkv_b,
           out_w, out_b, p1_w, p1_b, p2_w, p2_b, edges_index):
    g = bn_gamma.reshape(1, D)
    b = bn_beta.reshape(1, D)
    hw = _tc_pre(x, g, b, gcn_W[0])
    edges_flat = edges_index.reshape(R * 2 * E)
    ew_flat = edges_weight.reshape(R * E)
    norm = _sc_prep(edges_flat, ew_flat)
    out = None
    for i in range(NL):
        agg = _sc_spmm(hw.reshape(R * N, D), edges_flat, norm)
        gb = gcn_b[i].reshape(1, D)
        qb = qkv_b[i].reshape(1, 3 * D)
        ob = out_b[i].reshape(1, D)
        if i < NL - 1:
            hw = _tc_attn(agg, gb, qkv_w[i], qb, out_w[i], ob, gcn_W[i + 1])
        else:
            out = _tc_final(agg, gb, qkv_w[i], qb, out_w[i], ob,
                            p1_w, p1_b.reshape(1, H1), p2_w, p2_b.reshape(1, H2))
    return out


# speedup vs baseline: 13.4934x; 1.0324x over previous
import functools

import jax
import jax.numpy as jnp
from jax import lax
from jax.experimental import pallas as pl
from jax.experimental.pallas import tpu as pltpu
from jax.experimental.pallas import tpu_sc as plsc

N = 10000
E = 320000
R = 2
D = 128
H = 8
NL = 3
H1 = 128
H2 = 64

NP = 10240
NT = 16
EPT = E // NT
ROWS_PT = NP // NT
K = 80
KD = 80
KN = 400

_mesh = plsc.VectorSubcoreMesh(core_axis_name="c", subcore_axis_name="s")


def _rsqrt16(d):
    i = lax.bitcast_convert_type(d, jnp.int32)
    y = lax.bitcast_convert_type(jnp.int32(0x5F3759DF) - (i >> 1), jnp.float32)
    for _ in range(3):
        y = y * (1.5 - 0.5 * d * y * y)
    return jnp.where(d > 0.0, y, 0.0)


@functools.partial(
    pl.kernel,
    out_type=jax.ShapeDtypeStruct((R * E,), jnp.float32),
    mesh=_mesh,
    compiler_params=pltpu.CompilerParams(needs_layout_passes=False),
    scratch_types=[
        pltpu.VMEM_SHARED((NP,), jnp.float32),
        pltpu.VMEM((ROWS_PT,), jnp.float32),
        pltpu.VMEM((KD,), jnp.int32),
        pltpu.VMEM((KD,), jnp.float32),
        pltpu.VMEM((NP,), jnp.float32),
        pltpu.VMEM((KN,), jnp.int32),
        pltpu.VMEM((KN,), jnp.int32),
        pltpu.VMEM((KN,), jnp.float32),
        pltpu.VMEM((KN,), jnp.float32),
    ],
)
def _sc_prep(edges, ew, norm_out, deg_sh, zv, dbuf, wbuf, disv, sb2, db2, wb2, nb2):
    r = lax.axis_index("c")
    s = lax.axis_index("s")
    src_base = r * (2 * E)
    dst_base = src_base + E
    w_base = r * E

    def z0(i, c):
        zv[pl.ds(i * 16, 16)] = jnp.zeros((16,), jnp.float32)
        return c
    lax.fori_loop(0, ROWS_PT // 16, z0, 0)
    pltpu.sync_copy(zv, deg_sh.at[pl.ds(pl.multiple_of(s * ROWS_PT, 8), ROWS_PT)])
    plsc.subcore_barrier()

    def body1(i, c):
        e0 = s * EPT + i * KD
        pltpu.sync_copy(edges.at[pl.ds(pl.multiple_of(dst_base + e0, 8), KD)], dbuf)
        pltpu.sync_copy(ew.at[pl.ds(pl.multiple_of(w_base + e0, 8), KD)], wbuf)
        pltpu.sync_copy(wbuf, deg_sh.at[dbuf], add=True)
        return c
    lax.fori_loop(0, EPT // KD, body1, 0)
    plsc.subcore_barrier()

    pltpu.sync_copy(deg_sh, disv)
    def body2(i, c):
        sl = pl.ds(i * 16, 16)
        disv[sl] = _rsqrt16(disv[sl])
        return c
    lax.fori_loop(0, NP // 16, body2, 0)

    def body3(i, c):
        e0 = s * EPT + i * KN
        pltpu.sync_copy(edges.at[pl.ds(pl.multiple_of(src_base + e0, 8), KN)], sb2)
        pltpu.sync_copy(edges.at[pl.ds(pl.multiple_of(dst_base + e0, 8), KN)], db2)
        pltpu.sync_copy(ew.at[pl.ds(pl.multiple_of(w_base + e0, 8), KN)], wb2)
        def inner(j, c2):
            sl = pl.ds(j * 16, 16)
            a = plsc.load_gather(disv, [sb2[sl]])
            b = plsc.load_gather(disv, [db2[sl]])
            nb2[sl] = a * b * wb2[sl]
            return c2
        lax.fori_loop(0, KN // 16, inner, 0)
        pltpu.sync_copy(nb2, norm_out.at[pl.ds(pl.multiple_of(w_base + e0, 8), KN)])
        return c
    lax.fori_loop(0, EPT // KN, body3, 0)


KS = 80
NC = EPT // KS
NB = 2
NM = 10


@functools.partial(
    pl.kernel,
    out_type=jax.ShapeDtypeStruct((R, N, D), jnp.float32),
    mesh=_mesh,
    compiler_params=pltpu.CompilerParams(needs_layout_passes=False),
    scratch_types=[
        pltpu.VMEM_SHARED((NP, D), jnp.float32),
        [pltpu.VMEM((KS, D), jnp.float32)] * NB,
        [pltpu.VMEM((KS,), jnp.int32)] * NM,
        [pltpu.VMEM((KS,), jnp.int32)] * NM,
        [pltpu.VMEM((KS,), jnp.float32)] * NM,
        [pltpu.SemaphoreType.DMA] * NB,
        [pltpu.SemaphoreType.DMA] * NM,
        pltpu.SemaphoreType.DMA,
    ],
)
def _sc_spmm(hw, edges, norm, agg, acc, rows, six, dix, nbf,
             sem_g, sem_m, sem_s):
    r = lax.axis_index("c")
    s = lax.axis_index("s")
    rbase = r * N
    src_base = r * (2 * E) + s * EPT
    dst_base = src_base + E
    w_base = r * E + s * EPT

    def zb(i, c):
        for v in range(8):
            rows[0][i, pl.ds(v * 16, 16)] = jnp.zeros((16,), jnp.float32)
        return c
    lax.fori_loop(0, KS, zb, 0)
    row0 = pl.multiple_of(s * ROWS_PT, 8)
    def zc(i, c):
        pltpu.sync_copy(rows[0], acc.at[pl.ds(row0 + i * KS, KS)])
        return c
    lax.fori_loop(0, ROWS_PT // KS, zc, 0)
    plsc.subcore_barrier()

    def meta(i, m):
        e0 = i * KS
        pltpu.async_copy(edges.at[pl.ds(pl.multiple_of(src_base + e0, 8), KS)],
                         six[m], sem_m[m])
        pltpu.async_copy(edges.at[pl.ds(pl.multiple_of(dst_base + e0, 8), KS)],
                         dix[m], sem_m[m])
        pltpu.async_copy(norm.at[pl.ds(pl.multiple_of(w_base + e0, 8), KS)],
                         nbf[m], sem_m[m])

    def wait_m(m):
        pltpu.make_async_copy(edges.at[pl.ds(0, KS)], six[m], sem_m[m]).wait()
        pltpu.make_async_copy(edges.at[pl.ds(0, KS)], dix[m], sem_m[m]).wait()
        pltpu.make_async_copy(norm.at[pl.ds(0, KS)], nbf[m], sem_m[m]).wait()

    def offset_src(m):
        for v in range(KS // 16):
            sl = pl.ds(v * 16, 16)
            six[m][sl] = six[m][sl] + rbase

    def wait_g(b):
        pltpu.make_async_copy(hw.at[pl.ds(0, KS)], rows[b], sem_g[b]).wait()

    def scale_scatter(t, last):
        b = t % NB
        wait_g(b)
        def scale(j, c2):
            nb = plsc.load_gather(nbf[t], [jnp.zeros((16,), jnp.int32) + j])
            for v in range(8):
                sl = pl.ds(v * 16, 16)
                rows[b][j, sl] = rows[b][j, sl] * nb
            return c2
        lax.fori_loop(0, KS, scale, 0, unroll=2)
        if last:
            pltpu.sync_copy(rows[b], acc.at[dix[t]], add=True)
            return None
        return pltpu.async_copy(rows[b], acc.at[dix[t]], sem_s, add=True)

    def next_gather(g, t):
        t1 = (t + 1) % NM
        wait_m(t1)
        offset_src(t1)
        pltpu.async_copy(hw.at[six[t1]], rows[(t + 1) % NB], sem_g[(t + 1) % NB])

    for t in range(NM):
        meta(t, t)
    wait_m(0)
    offset_src(0)
    pltpu.async_copy(hw.at[six[0]], rows[0], sem_g[0])

    def outer(g, c):
        pend = None
        for t in range(NM):
            if pend is not None:
                pend.wait()
            next_gather(g, t)
            pend = scale_scatter(t, last=(t == NM - 1))
            meta((g + 1) * NM + t, t)
        return c
    lax.fori_loop(0, NC // NM - 1, outer, 0)
    gl = NC // NM - 1
    pend = None
    for t in range(NM):
        if pend is not None:
            pend.wait()
        if t < NM - 1:
            next_gather(gl, t)
        pend = scale_scatter(t, last=(t == NM - 1))
    plsc.subcore_barrier()

    @pl.when(s < NT - 1)
    def _():
        pltpu.sync_copy(acc.at[pl.ds(row0, ROWS_PT)], agg.at[r, pl.ds(row0, ROWS_PT)])
    @pl.when(s == NT - 1)
    def _():
        tail = N - (NT - 1) * ROWS_PT
        base = pl.multiple_of((NT - 1) * ROWS_PT, 8)
        pltpu.sync_copy(acc.at[pl.ds(base, tail)], agg.at[r, pl.ds(base, tail)])


def _dot_t(a, w):
    return lax.dot_general(a, w, (((1,), (1,)), ((), ())),
                           preferred_element_type=jnp.float32)


def _head_matrix():
    lane = lax.broadcasted_iota(jnp.int32, (D, H), 0)
    hh = lax.broadcasted_iota(jnp.int32, (D, H), 1)
    return (lane // (D // H) == hh).astype(jnp.float32)


def _tc_pre_body(x_ref, g_ref, b_ref, w0_ref, hw_ref):
    x = x_ref[...]
    s1 = jnp.sum(x, axis=0, keepdims=True)
    s2 = jnp.sum(x * x, axis=0, keepdims=True)
    mean = s1 / N
    var = s2 / N - mean * mean
    xn = (x - mean) * lax.rsqrt(var + 1e-5) * g_ref[...] + b_ref[...]
    hw0 = _dot_t(xn, w0_ref[...])
    hw_ref[0] = hw0
    hw_ref[1] = hw0


def _attn_core(agg_ref, gb_ref, qkvw_ref, qkvb_ref, ow_ref, ob_ref):
    gb = gb_ref[...]
    z0 = agg_ref[0] + gb
    z1 = agg_ref[1] + gb
    qkvw = qkvw_ref[...]
    qkvb = qkvb_ref[...]
    wq, wk, wv = qkvw[:D], qkvw[D:2 * D], qkvw[2 * D:]
    bq, bk, bv = qkvb[:, :D], qkvb[:, D:2 * D], qkvb[:, 2 * D:]
    q0 = _dot_t(z0, wq) + bq
    q1 = _dot_t(z1, wq) + bq
    k0 = _dot_t(z0, wk) + bk
    k1 = _dot_t(z1, wk) + bk
    v0 = _dot_t(z0, wv) + bv
    v1 = _dot_t(z1, wv) + bv

    M = _head_matrix()
    scale = 1.0 / jnp.sqrt(jnp.float32(D // H))
    def hsum(t):
        return lax.dot_general(t, M, (((1,), (0,)), ((), ())),
                               preferred_element_type=jnp.float32)
    s00 = hsum(q0 * k0) * scale
    s01 = hsum(q0 * k1) * scale
    s10 = hsum(q1 * k0) * scale
    s11 = hsum(q1 * k1) * scale

    m0 = jnp.maximum(s00, s01)
    e00 = jnp.exp(s00 - m0)
    e01 = jnp.exp(s01 - m0)
    a00 = e00 / (e00 + e01)
    a01 = e01 / (e00 + e01)
    m1 = jnp.maximum(s10, s11)
    e10 = jnp.exp(s10 - m1)
    e11 = jnp.exp(s11 - m1)
    a10 = e10 / (e10 + e11)
    a11 = e11 / (e10 + e11)

    def hexp(a):
        return lax.dot_general(a, M, (((1,), (1,)), ((), ())),
                               preferred_element_type=jnp.float32)
    o0 = hexp(a00) * v0 + hexp(a01) * v1
    o1 = hexp(a10) * v0 + hexp(a11) * v1
    ob = ob_ref[...]
    r0 = jnp.maximum(_dot_t(o0, ow_ref[...]) + ob, 0.0)
    r1 = jnp.maximum(_dot_t(o1, ow_ref[...]) + ob, 0.0)
    return r0, r1


def _tc_attn_body(agg_ref, gb_ref, qkvw_ref, qkvb_ref, ow_ref, ob_ref, wn_ref, out_ref):
    r0, r1 = _attn_core(agg_ref, gb_ref, qkvw_ref, qkvb_ref, ow_ref, ob_ref)
    wn = wn_ref[...]
    out_ref[0] = _dot_t(r0, wn)
    out_ref[1] = _dot_t(r1, wn)


def _tc_final_body(agg_ref, gb_ref, qkvw_ref, qkvb_ref, ow_ref, ob_ref,
                   p1w_ref, p1b_ref, p2w_ref, p2b_ref, out_ref):
    r0, r1 = _attn_core(agg_ref, gb_ref, qkvw_ref, qkvb_ref, ow_ref, ob_ref)
    p1w, p1b = p1w_ref[...], p1b_ref[...]
    p2w, p2b = p2w_ref[...], p2b_ref[...]
    def mlp(t):
        h = _dot_t(t, p1w) + p1b
        h = jnp.where(h > 0, h, 0.01 * h)
        h = _dot_t(h, p2w) + p2b
        return jnp.where(h > 0, h, 0.01 * h)
    out_ref[0] = mlp(r0)
    out_ref[1] = mlp(r1)


_BN = 1000


def _full(shape):
    return pl.BlockSpec(shape, lambda i: tuple(0 for _ in shape))


def _tc_pre(x, g, b, w0):
    return pl.pallas_call(
        _tc_pre_body,
        out_shape=jax.ShapeDtypeStruct((R, N, D), jnp.float32),
    )(x, g, b, w0)


def _tc_attn(agg, gb, qkvw, qkvb, ow, ob, wn):
    grid = (N // _BN,)
    return pl.pallas_call(
        _tc_attn_body,
        grid=grid,
        in_specs=[
            pl.BlockSpec((R, _BN, D), lambda i: (0, i, 0)),
            _full((1, D)), _full((3 * D, D)), _full((1, 3 * D)),
            _full((D, D)), _full((1, D)), _full((D, D)),
        ],
        out_specs=pl.BlockSpec((R, _BN, D), lambda i: (0, i, 0)),
        out_shape=jax.ShapeDtypeStruct((R, N, D), jnp.float32),
    )(agg, gb, qkvw, qkvb, ow, ob, wn)


def _tc_final(agg, gb, qkvw, qkvb, ow, ob, p1w, p1b, p2w, p2b):
    grid = (N // _BN,)
    return pl.pallas_call(
        _tc_final_body,
        grid=grid,
        in_specs=[
            pl.BlockSpec((R, _BN, D), lambda i: (0, i, 0)),
            _full((1, D)), _full((3 * D, D)), _full((1, 3 * D)),
            _full((D, D)), _full((1, D)),
            _full((H1, D)), _full((1, H1)), _full((H2, H1)), _full((1, H2)),
        ],
        out_specs=pl.BlockSpec((R, _BN, H2), lambda i: (0, i, 0)),
        out_shape=jax.ShapeDtypeStruct((R, N, H2), jnp.float32),
    )(agg, gb, qkvw, qkvb, ow, ob, p1w, p1b, p2w, p2b)


def kernel(x, edges_weight, bn_gamma, bn_beta, gcn_W, gcn_b, qkv_w, qkv_b,
           out_w, out_b, p1_w, p1_b, p2_w, p2_b, edges_index):
    g = bn_gamma.reshape(1, D)
    b = bn_beta.reshape(1, D)
    hw = _tc_pre(x, g, b, gcn_W[0])
    edges_flat = edges_index.reshape(R * 2 * E)
    ew_flat = edges_weight.reshape(R * E)
    norm = _sc_prep(edges_flat, ew_flat)
    out = None
    for i in range(NL):
        agg = _sc_spmm(hw.reshape(R * N, D), edges_flat, norm)
        gb = gcn_b[i].reshape(1, D)
        qb = qkv_b[i].reshape(1, 3 * D)
        ob = out_b[i].reshape(1, D)
        if i < NL - 1:
            hw = _tc_attn(agg, gb, qkv_w[i], qb, out_w[i], ob, gcn_W[i + 1])
        else:
            out = _tc_final(agg, gb, qkv_w[i], qb, out_w[i], ob,
                            p1_w, p1_b.reshape(1, H1), p2_w, p2_b.reshape(1, H2))
    return out


# speedup vs baseline: 16.8216x; 1.2466x over previous
import functools

import jax
import jax.numpy as jnp
from jax import lax
from jax.experimental import pallas as pl
from jax.experimental.pallas import tpu as pltpu
from jax.experimental.pallas import tpu_sc as plsc

N = 10000
E = 320000
R = 2
D = 128
H = 8
NL = 3
H1 = 128
H2 = 64

NP = 10240
NT = 16
EPT = E // NT
ROWS_PT = NP // NT
K = 80
KD = 80
KN = 400

_mesh = plsc.VectorSubcoreMesh(core_axis_name="c", subcore_axis_name="s")


def _rsqrt16(d):
    i = lax.bitcast_convert_type(d, jnp.int32)
    y = lax.bitcast_convert_type(jnp.int32(0x5F3759DF) - (i >> 1), jnp.float32)
    for _ in range(3):
        y = y * (1.5 - 0.5 * d * y * y)
    return jnp.where(d > 0.0, y, 0.0)


ND1 = 10
ND3 = 5


@functools.partial(
    pl.kernel,
    out_type=jax.ShapeDtypeStruct((R * E,), jnp.float32),
    mesh=_mesh,
    compiler_params=pltpu.CompilerParams(needs_layout_passes=False),
    scratch_types=[
        pltpu.VMEM_SHARED((NP,), jnp.float32),
        pltpu.VMEM((ROWS_PT,), jnp.float32),
        [pltpu.VMEM((KD,), jnp.int32)] * ND1,
        [pltpu.VMEM((KD,), jnp.float32)] * ND1,
        pltpu.VMEM((NP,), jnp.float32),
        [pltpu.VMEM((KN,), jnp.int32)] * ND3,
        [pltpu.VMEM((KN,), jnp.int32)] * ND3,
        [pltpu.VMEM((KN,), jnp.float32)] * ND3,
        [pltpu.SemaphoreType.DMA] * ND1,
        [pltpu.SemaphoreType.DMA] * ND3,
        [pltpu.SemaphoreType.DMA] * ND1,
        [pltpu.SemaphoreType.DMA] * ND3,
    ],
)
def _sc_prep(edges, ew, norm_out, deg_sh, zv, dbuf, wbuf, disv, sb2, db2, wb2,
             sem1, sem3, sem1s, sem3w):
    r = lax.axis_index("c")
    s = lax.axis_index("s")
    src_base = r * (2 * E) + s * EPT
    dst_base = src_base + E
    w_base = r * E + s * EPT

    def z0(i, c):
        zv[pl.ds(i * 16, 16)] = jnp.zeros((16,), jnp.float32)
        return c
    lax.fori_loop(0, ROWS_PT // 16, z0, 0)
    pltpu.sync_copy(zv, deg_sh.at[pl.ds(pl.multiple_of(s * ROWS_PT, 8), ROWS_PT)])
    plsc.subcore_barrier()

    NC1 = EPT // KD
    def meta1(i, m):
        e0 = i * KD
        pltpu.async_copy(edges.at[pl.ds(pl.multiple_of(dst_base + e0, 8), KD)],
                         dbuf[m], sem1[m])
        pltpu.async_copy(ew.at[pl.ds(pl.multiple_of(w_base + e0, 8), KD)],
                         wbuf[m], sem1[m])
    def wait1(m):
        pltpu.make_async_copy(edges.at[pl.ds(0, KD)], dbuf[m], sem1[m]).wait()
        pltpu.make_async_copy(ew.at[pl.ds(0, KD)], wbuf[m], sem1[m]).wait()
    for t in range(ND1):
        meta1(t, t)
    def outer1(g, c):
        pends = []
        for t in range(ND1):
            wait1(t)
            pends.append(
                pltpu.async_copy(wbuf[t], deg_sh.at[dbuf[t]], sem1s[t], add=True))
        for t in range(ND1):
            pends[t].wait()
            meta1((g + 1) * ND1 + t, t)
        return c
    lax.fori_loop(0, NC1 // ND1 - 1, outer1, 0)
    pends = []
    for t in range(ND1):
        wait1(t)
        pends.append(
            pltpu.async_copy(wbuf[t], deg_sh.at[dbuf[t]], sem1s[t], add=True))
    for t in range(ND1):
        pends[t].wait()
    plsc.subcore_barrier()

    pltpu.sync_copy(deg_sh, disv)
    def body2(i, c):
        sl = pl.ds(i * 16, 16)
        disv[sl] = _rsqrt16(disv[sl])
        return c
    lax.fori_loop(0, NP // 16, body2, 0)

    NC3 = EPT // KN
    def meta3(i, m):
        e0 = i * KN
        pltpu.async_copy(edges.at[pl.ds(pl.multiple_of(src_base + e0, 8), KN)],
                         sb2[m], sem3[m])
        pltpu.async_copy(edges.at[pl.ds(pl.multiple_of(dst_base + e0, 8), KN)],
                         db2[m], sem3[m])
        pltpu.async_copy(ew.at[pl.ds(pl.multiple_of(w_base + e0, 8), KN)],
                         wb2[m], sem3[m])
    def wait3(m):
        pltpu.make_async_copy(edges.at[pl.ds(0, KN)], sb2[m], sem3[m]).wait()
        pltpu.make_async_copy(edges.at[pl.ds(0, KN)], db2[m], sem3[m]).wait()
        pltpu.make_async_copy(ew.at[pl.ds(0, KN)], wb2[m], sem3[m]).wait()
    def norm_chunk(i, t):
        wait3(t)
        def inner(j, c2):
            sl = pl.ds(j * 16, 16)
            a = plsc.load_gather(disv, [sb2[t][sl]])
            b = plsc.load_gather(disv, [db2[t][sl]])
            wb2[t][sl] = a * b * wb2[t][sl]
            return c2
        lax.fori_loop(0, KN // 16, inner, 0)
        e0 = i * KN
        return pltpu.async_copy(
            wb2[t], norm_out.at[pl.ds(pl.multiple_of(w_base + e0, 8), KN)],
            sem3w[t])
    for t in range(ND3):
        meta3(t, t)
    def outer3(g, c):
        pends = []
        for t in range(ND3):
            pends.append(norm_chunk(g * ND3 + t, t))
        for t in range(ND3):
            pends[t].wait()
            meta3((g + 1) * ND3 + t, t)
        return c
    lax.fori_loop(0, NC3 // ND3 - 1, outer3, 0)
    pends = []
    for t in range(ND3):
        pends.append(norm_chunk((NC3 // ND3 - 1) * ND3 + t, t))
    for t in range(ND3):
        pends[t].wait()


KS = 80
NC = EPT // KS
NB = 2
NM = 10


@functools.partial(
    pl.kernel,
    out_type=jax.ShapeDtypeStruct((R, N, D), jnp.float32),
    mesh=_mesh,
    compiler_params=pltpu.CompilerParams(needs_layout_passes=False),
    scratch_types=[
        pltpu.VMEM_SHARED((NP, D), jnp.float32),
        [pltpu.VMEM((KS, D), jnp.float32)] * NB,
        [pltpu.VMEM((KS,), jnp.int32)] * NM,
        [pltpu.VMEM((KS,), jnp.int32)] * NM,
        [pltpu.VMEM((KS,), jnp.float32)] * NM,
        [pltpu.SemaphoreType.DMA] * NB,
        [pltpu.SemaphoreType.DMA] * NM,
        pltpu.SemaphoreType.DMA,
    ],
)
def _sc_spmm(hw, edges, norm, agg, acc, rows, six, dix, nbf,
             sem_g, sem_m, sem_s):
    r = lax.axis_index("c")
    s = lax.axis_index("s")
    rbase = r * N
    src_base = r * (2 * E) + s * EPT
    dst_base = src_base + E
    w_base = r * E + s * EPT

    def zb(i, c):
        for v in range(8):
            rows[0][i, pl.ds(v * 16, 16)] = jnp.zeros((16,), jnp.float32)
        return c
    lax.fori_loop(0, KS, zb, 0)
    row0 = pl.multiple_of(s * ROWS_PT, 8)
    def zc(i, c):
        pltpu.sync_copy(rows[0], acc.at[pl.ds(row0 + i * KS, KS)])
        return c
    lax.fori_loop(0, ROWS_PT // KS, zc, 0)
    plsc.subcore_barrier()

    def meta(i, m):
        e0 = i * KS
        pltpu.async_copy(edges.at[pl.ds(pl.multiple_of(src_base + e0, 8), KS)],
                         six[m], sem_m[m])
        pltpu.async_copy(edges.at[pl.ds(pl.multiple_of(dst_base + e0, 8), KS)],
                         dix[m], sem_m[m])
        pltpu.async_copy(norm.at[pl.ds(pl.multiple_of(w_base + e0, 8), KS)],
                         nbf[m], sem_m[m])

    def wait_m(m):
        pltpu.make_async_copy(edges.at[pl.ds(0, KS)], six[m], sem_m[m]).wait()
        pltpu.make_async_copy(edges.at[pl.ds(0, KS)], dix[m], sem_m[m]).wait()
        pltpu.make_async_copy(norm.at[pl.ds(0, KS)], nbf[m], sem_m[m]).wait()

    def offset_src(m):
        for v in range(KS // 16):
            sl = pl.ds(v * 16, 16)
            six[m][sl] = six[m][sl] + rbase

    def wait_g(b):
        pltpu.make_async_copy(hw.at[pl.ds(0, KS)], rows[b], sem_g[b]).wait()

    def scale_scatter(t, last):
        b = t % NB
        wait_g(b)
        def scale(j, c2):
            nb = plsc.load_gather(nbf[t], [jnp.zeros((16,), jnp.int32) + j])
            for v in range(8):
                sl = pl.ds(v * 16, 16)
                rows[b][j, sl] = rows[b][j, sl] * nb
            return c2
        lax.fori_loop(0, KS, scale, 0, unroll=2)
        if last:
            pltpu.sync_copy(rows[b], acc.at[dix[t]], add=True)
            return None
        return pltpu.async_copy(rows[b], acc.at[dix[t]], sem_s, add=True)

    def next_gather(g, t):
        t1 = (t + 1) % NM
        wait_m(t1)
        offset_src(t1)
        pltpu.async_copy(hw.at[six[t1]], rows[(t + 1) % NB], sem_g[(t + 1) % NB])

    for t in range(NM):
        meta(t, t)
    wait_m(0)
    offset_src(0)
    pltpu.async_copy(hw.at[six[0]], rows[0], sem_g[0])

    def outer(g, c):
        pend = None
        for t in range(NM):
            if pend is not None:
                pend.wait()
            next_gather(g, t)
            pend = scale_scatter(t, last=(t == NM - 1))
            meta((g + 1) * NM + t, t)
        return c
    lax.fori_loop(0, NC // NM - 1, outer, 0)
    gl = NC // NM - 1
    pend = None
    for t in range(NM):
        if pend is not None:
            pend.wait()
        if t < NM - 1:
            next_gather(gl, t)
        pend = scale_scatter(t, last=(t == NM - 1))
    plsc.subcore_barrier()

    @pl.when(s < NT - 1)
    def _():
        pltpu.sync_copy(acc.at[pl.ds(row0, ROWS_PT)], agg.at[r, pl.ds(row0, ROWS_PT)])
    @pl.when(s == NT - 1)
    def _():
        tail = N - (NT - 1) * ROWS_PT
        base = pl.multiple_of((NT - 1) * ROWS_PT, 8)
        pltpu.sync_copy(acc.at[pl.ds(base, tail)], agg.at[r, pl.ds(base, tail)])


def _dot_t(a, w):
    return lax.dot_general(a, w, (((1,), (1,)), ((), ())),
                           preferred_element_type=jnp.float32)


def _head_matrix():
    lane = lax.broadcasted_iota(jnp.int32, (D, H), 0)
    hh = lax.broadcasted_iota(jnp.int32, (D, H), 1)
    return (lane // (D // H) == hh).astype(jnp.float32)


def _tc_pre_body(x_ref, g_ref, b_ref, w0_ref, hw_ref):
    x = x_ref[...]
    s1 = jnp.sum(x, axis=0, keepdims=True)
    s2 = jnp.sum(x * x, axis=0, keepdims=True)
    mean = s1 / N
    var = s2 / N - mean * mean
    xn = (x - mean) * lax.rsqrt(var + 1e-5) * g_ref[...] + b_ref[...]
    hw0 = _dot_t(xn, w0_ref[...])
    hw_ref[0] = hw0
    hw_ref[1] = hw0


def _attn_core(agg_ref, gb_ref, qkvw_ref, qkvb_ref, ow_ref, ob_ref):
    gb = gb_ref[...]
    z0 = agg_ref[0] + gb
    z1 = agg_ref[1] + gb
    qkvw = qkvw_ref[...]
    qkvb = qkvb_ref[...]
    wq, wk, wv = qkvw[:D], qkvw[D:2 * D], qkvw[2 * D:]
    bq, bk, bv = qkvb[:, :D], qkvb[:, D:2 * D], qkvb[:, 2 * D:]
    q0 = _dot_t(z0, wq) + bq
    q1 = _dot_t(z1, wq) + bq
    k0 = _dot_t(z0, wk) + bk
    k1 = _dot_t(z1, wk) + bk
    v0 = _dot_t(z0, wv) + bv
    v1 = _dot_t(z1, wv) + bv

    M = _head_matrix()
    scale = 1.0 / jnp.sqrt(jnp.float32(D // H))
    def hsum(t):
        return lax.dot_general(t, M, (((1,), (0,)), ((), ())),
                               preferred_element_type=jnp.float32)
    s00 = hsum(q0 * k0) * scale
    s01 = hsum(q0 * k1) * scale
    s10 = hsum(q1 * k0) * scale
    s11 = hsum(q1 * k1) * scale

    m0 = jnp.maximum(s00, s01)
    e00 = jnp.exp(s00 - m0)
    e01 = jnp.exp(s01 - m0)
    a00 = e00 / (e00 + e01)
    a01 = e01 / (e00 + e01)
    m1 = jnp.maximum(s10, s11)
    e10 = jnp.exp(s10 - m1)
    e11 = jnp.exp(s11 - m1)
    a10 = e10 / (e10 + e11)
    a11 = e11 / (e10 + e11)

    def hexp(a):
        return lax.dot_general(a, M, (((1,), (1,)), ((), ())),
                               preferred_element_type=jnp.float32)
    o0 = hexp(a00) * v0 + hexp(a01) * v1
    o1 = hexp(a10) * v0 + hexp(a11) * v1
    ob = ob_ref[...]
    r0 = jnp.maximum(_dot_t(o0, ow_ref[...]) + ob, 0.0)
    r1 = jnp.maximum(_dot_t(o1, ow_ref[...]) + ob, 0.0)
    return r0, r1


def _tc_attn_body(agg_ref, gb_ref, qkvw_ref, qkvb_ref, ow_ref, ob_ref, wn_ref, out_ref):
    r0, r1 = _attn_core(agg_ref, gb_ref, qkvw_ref, qkvb_ref, ow_ref, ob_ref)
    wn = wn_ref[...]
    out_ref[0] = _dot_t(r0, wn)
    out_ref[1] = _dot_t(r1, wn)


def _tc_final_body(agg_ref, gb_ref, qkvw_ref, qkvb_ref, ow_ref, ob_ref,
                   p1w_ref, p1b_ref, p2w_ref, p2b_ref, out_ref):
    r0, r1 = _attn_core(agg_ref, gb_ref, qkvw_ref, qkvb_ref, ow_ref, ob_ref)
    p1w, p1b = p1w_ref[...], p1b_ref[...]
    p2w, p2b = p2w_ref[...], p2b_ref[...]
    def mlp(t):
        h = _dot_t(t, p1w) + p1b
        h = jnp.where(h > 0, h, 0.01 * h)
        h = _dot_t(h, p2w) + p2b
        return jnp.where(h > 0, h, 0.01 * h)
    out_ref[0] = mlp(r0)
    out_ref[1] = mlp(r1)


_BN = 1000


def _full(shape):
    return pl.BlockSpec(shape, lambda i: tuple(0 for _ in shape))


def _tc_pre(x, g, b, w0):
    return pl.pallas_call(
        _tc_pre_body,
        out_shape=jax.ShapeDtypeStruct((R, N, D), jnp.float32),
    )(x, g, b, w0)


def _tc_attn(agg, gb, qkvw, qkvb, ow, ob, wn):
    grid = (N // _BN,)
    return pl.pallas_call(
        _tc_attn_body,
        grid=grid,
        in_specs=[
            pl.BlockSpec((R, _BN, D), lambda i: (0, i, 0)),
            _full((1, D)), _full((3 * D, D)), _full((1, 3 * D)),
            _full((D, D)), _full((1, D)), _full((D, D)),
        ],
        out_specs=pl.BlockSpec((R, _BN, D), lambda i: (0, i, 0)),
        out_shape=jax.ShapeDtypeStruct((R, N, D), jnp.float32),
    )(agg, gb, qkvw, qkvb, ow, ob, wn)


def _tc_final(agg, gb, qkvw, qkvb, ow, ob, p1w, p1b, p2w, p2b):
    grid = (N // _BN,)
    return pl.pallas_call(
        _tc_final_body,
        grid=grid,
        in_specs=[
            pl.BlockSpec((R, _BN, D), lambda i: (0, i, 0)),
            _full((1, D)), _full((3 * D, D)), _full((1, 3 * D)),
            _full((D, D)), _full((1, D)),
            _full((H1, D)), _full((1, H1)), _full((H2, H1)), _full((1, H2)),
        ],
        out_specs=pl.BlockSpec((R, _BN, H2), lambda i: (0, i, 0)),
        out_shape=jax.ShapeDtypeStruct((R, N, H2), jnp.float32),
    )(agg, gb, qkvw, qkvb, ow, ob, p1w, p1b, p2w, p2b)


def kernel(x, edges_weight, bn_gamma, bn_beta, gcn_W, gcn_b, qkv_w, qkv_b,
           out_w, out_b, p1_w, p1_b, p2_w, p2_b, edges_index):
    g = bn_gamma.reshape(1, D)
    b = bn_beta.reshape(1, D)
    hw = _tc_pre(x, g, b, gcn_W[0])
    edges_flat = edges_index.reshape(R * 2 * E)
    ew_flat = edges_weight.reshape(R * E)
    norm = _sc_prep(edges_flat, ew_flat)
    out = None
    for i in range(NL):
        agg = _sc_spmm(hw.reshape(R * N, D), edges_flat, norm)
        gb = gcn_b[i].reshape(1, D)
        qb = qkv_b[i].reshape(1, 3 * D)
        ob = out_b[i].reshape(1, D)
        if i < NL - 1:
            hw = _tc_attn(agg, gb, qkv_w[i], qb, out_w[i], ob, gcn_W[i + 1])
        else:
            out = _tc_final(agg, gb, qkv_w[i], qb, out_w[i], ob,
                            p1_w, p1_b.reshape(1, H1), p2_w, p2_b.reshape(1, H2))
    return out


# speedup vs baseline: 16.8342x; 1.0008x over previous
import functools

import jax
import jax.numpy as jnp
from jax import lax
from jax.experimental import pallas as pl
from jax.experimental.pallas import tpu as pltpu
from jax.experimental.pallas import tpu_sc as plsc

N = 10000
E = 320000
R = 2
D = 128
H = 8
NL = 3
H1 = 128
H2 = 64

NP = 10240
NT = 16
EPT = E // NT
ROWS_PT = NP // NT
K = 80
KD = 80
KN = 400

_mesh = plsc.VectorSubcoreMesh(core_axis_name="c", subcore_axis_name="s")


def _rsqrt16(d):
    i = lax.bitcast_convert_type(d, jnp.int32)
    y = lax.bitcast_convert_type(jnp.int32(0x5F3759DF) - (i >> 1), jnp.float32)
    for _ in range(3):
        y = y * (1.5 - 0.5 * d * y * y)
    return jnp.where(d > 0.0, y, 0.0)


ND1 = 10
ND3 = 5


@functools.partial(
    pl.kernel,
    out_type=jax.ShapeDtypeStruct((R * E,), jnp.float32),
    mesh=_mesh,
    compiler_params=pltpu.CompilerParams(needs_layout_passes=False),
    scratch_types=[
        pltpu.VMEM_SHARED((NP,), jnp.float32),
        pltpu.VMEM((ROWS_PT,), jnp.float32),
        [pltpu.VMEM((KD,), jnp.int32)] * ND1,
        [pltpu.VMEM((KD,), jnp.float32)] * ND1,
        pltpu.VMEM((NP,), jnp.float32),
        [pltpu.VMEM((KN,), jnp.int32)] * ND3,
        [pltpu.VMEM((KN,), jnp.int32)] * ND3,
        [pltpu.VMEM((KN,), jnp.float32)] * ND3,
        [pltpu.SemaphoreType.DMA] * ND1,
        [pltpu.SemaphoreType.DMA] * ND3,
        [pltpu.SemaphoreType.DMA] * ND1,
        [pltpu.SemaphoreType.DMA] * ND3,
    ],
)
def _sc_prep(edges, ew, norm_out, deg_sh, zv, dbuf, wbuf, disv, sb2, db2, wb2,
             sem1, sem3, sem1s, sem3w):
    r = lax.axis_index("c")
    s = lax.axis_index("s")
    src_base = r * (2 * E) + s * EPT
    dst_base = src_base + E
    w_base = r * E + s * EPT

    def z0(i, c):
        zv[pl.ds(i * 16, 16)] = jnp.zeros((16,), jnp.float32)
        return c
    lax.fori_loop(0, ROWS_PT // 16, z0, 0)
    pltpu.sync_copy(zv, deg_sh.at[pl.ds(pl.multiple_of(s * ROWS_PT, 8), ROWS_PT)])
    plsc.subcore_barrier()

    NC1 = EPT // KD
    def meta1(i, m):
        e0 = i * KD
        pltpu.async_copy(edges.at[pl.ds(pl.multiple_of(dst_base + e0, 8), KD)],
                         dbuf[m], sem1[m])
        pltpu.async_copy(ew.at[pl.ds(pl.multiple_of(w_base + e0, 8), KD)],
                         wbuf[m], sem1[m])
    def wait1(m):
        pltpu.make_async_copy(edges.at[pl.ds(0, KD)], dbuf[m], sem1[m]).wait()
        pltpu.make_async_copy(ew.at[pl.ds(0, KD)], wbuf[m], sem1[m]).wait()
    for t in range(ND1):
        meta1(t, t)
    def outer1(g, c):
        pends = []
        for t in range(ND1):
            wait1(t)
            pends.append(
                pltpu.async_copy(wbuf[t], deg_sh.at[dbuf[t]], sem1s[t], add=True))
        for t in range(ND1):
            pends[t].wait()
            meta1((g + 1) * ND1 + t, t)
        return c
    lax.fori_loop(0, NC1 // ND1 - 1, outer1, 0)
    pends = []
    for t in range(ND1):
        wait1(t)
        pends.append(
            pltpu.async_copy(wbuf[t], deg_sh.at[dbuf[t]], sem1s[t], add=True))
    for t in range(ND1):
        pends[t].wait()
    plsc.subcore_barrier()

    pltpu.sync_copy(deg_sh, disv)
    def body2(i, c):
        sl = pl.ds(i * 16, 16)
        disv[sl] = _rsqrt16(disv[sl])
        return c
    lax.fori_loop(0, NP // 16, body2, 0)

    NC3 = EPT // KN
    def meta3(i, m):
        e0 = i * KN
        pltpu.async_copy(edges.at[pl.ds(pl.multiple_of(src_base + e0, 8), KN)],
                         sb2[m], sem3[m])
        pltpu.async_copy(edges.at[pl.ds(pl.multiple_of(dst_base + e0, 8), KN)],
                         db2[m], sem3[m])
        pltpu.async_copy(ew.at[pl.ds(pl.multiple_of(w_base + e0, 8), KN)],
                         wb2[m], sem3[m])
    def wait3(m):
        pltpu.make_async_copy(edges.at[pl.ds(0, KN)], sb2[m], sem3[m]).wait()
        pltpu.make_async_copy(edges.at[pl.ds(0, KN)], db2[m], sem3[m]).wait()
        pltpu.make_async_copy(ew.at[pl.ds(0, KN)], wb2[m], sem3[m]).wait()
    def norm_chunk(i, t):
        wait3(t)
        def inner(j, c2):
            sl = pl.ds(j * 16, 16)
            a = plsc.load_gather(disv, [sb2[t][sl]])
            b = plsc.load_gather(disv, [db2[t][sl]])
            wb2[t][sl] = a * b * wb2[t][sl]
            return c2
        lax.fori_loop(0, KN // 16, inner, 0)
        e0 = i * KN
        return pltpu.async_copy(
            wb2[t], norm_out.at[pl.ds(pl.multiple_of(w_base + e0, 8), KN)],
            sem3w[t])
    for t in range(ND3):
        meta3(t, t)
    def outer3(g, c):
        pends = []
        for t in range(ND3):
            pends.append(norm_chunk(g * ND3 + t, t))
        for t in range(ND3):
            pends[t].wait()
            meta3((g + 1) * ND3 + t, t)
        return c
    lax.fori_loop(0, NC3 // ND3 - 1, outer3, 0)
    pends = []
    for t in range(ND3):
        pends.append(norm_chunk((NC3 // ND3 - 1) * ND3 + t, t))
    for t in range(ND3):
        pends[t].wait()


KS = 80
NC = EPT // KS
NB = 2
NM = 10


@functools.partial(
    pl.kernel,
    out_type=jax.ShapeDtypeStruct((R, N, D), jnp.float32),
    mesh=_mesh,
    compiler_params=pltpu.CompilerParams(needs_layout_passes=False),
    scratch_types=[
        pltpu.VMEM_SHARED((NP, D), jnp.float32),
        [pltpu.VMEM((KS, D), jnp.float32)] * NB,
        [pltpu.VMEM((KS,), jnp.int32)] * NM,
        [pltpu.VMEM((KS,), jnp.int32)] * NM,
        [pltpu.VMEM((KS,), jnp.float32)] * NM,
        [pltpu.SemaphoreType.DMA] * NB,
        [pltpu.SemaphoreType.DMA] * NM,
        pltpu.SemaphoreType.DMA,
    ],
)
def _sc_spmm(hw, edges, norm, agg, acc, rows, six, dix, nbf,
             sem_g, sem_m, sem_s):
    r = lax.axis_index("c")
    s = lax.axis_index("s")
    rbase = r * N
    src_base = r * (2 * E) + s * EPT
    dst_base = src_base + E
    w_base = r * E + s * EPT

    def zb(i, c):
        for v in range(8):
            rows[0][i, pl.ds(v * 16, 16)] = jnp.zeros((16,), jnp.float32)
        return c
    lax.fori_loop(0, KS, zb, 0)
    row0 = pl.multiple_of(s * ROWS_PT, 8)
    def zc(i, c):
        pltpu.sync_copy(rows[0], acc.at[pl.ds(row0 + i * KS, KS)])
        return c
    lax.fori_loop(0, ROWS_PT // KS, zc, 0)
    plsc.subcore_barrier()

    def meta(i, m):
        e0 = i * KS
        pltpu.async_copy(edges.at[pl.ds(pl.multiple_of(src_base + e0, 8), KS)],
                         six[m], sem_m[m])
        pltpu.async_copy(edges.at[pl.ds(pl.multiple_of(dst_base + e0, 8), KS)],
                         dix[m], sem_m[m])
        pltpu.async_copy(norm.at[pl.ds(pl.multiple_of(w_base + e0, 8), KS)],
                         nbf[m], sem_m[m])

    def wait_m(m):
        pltpu.make_async_copy(edges.at[pl.ds(0, KS)], six[m], sem_m[m]).wait()
        pltpu.make_async_copy(edges.at[pl.ds(0, KS)], dix[m], sem_m[m]).wait()
        pltpu.make_async_copy(norm.at[pl.ds(0, KS)], nbf[m], sem_m[m]).wait()

    def offset_src(m):
        for v in range(KS // 16):
            sl = pl.ds(v * 16, 16)
            six[m][sl] = six[m][sl] + rbase

    def wait_g(b):
        pltpu.make_async_copy(hw.at[pl.ds(0, KS)], rows[b], sem_g[b]).wait()

    def scale_scatter(t):
        b = t % NB
        wait_g(b)
        def scale(j, c2):
            nb = plsc.load_gather(nbf[t], [jnp.zeros((16,), jnp.int32) + j])
            for v in range(8):
                sl = pl.ds(v * 16, 16)
                rows[b][j, sl] = rows[b][j, sl] * nb
            return c2
        lax.fori_loop(0, KS, scale, 0, unroll=2)
        return pltpu.async_copy(rows[b], acc.at[dix[t]], sem_s, add=True)

    def wait_s_boundary():
        pltpu.make_async_copy(rows[(NM - 1) % NB], acc.at[pl.ds(0, KS)],
                              sem_s).wait()

    def next_gather(g, t):
        t1 = (t + 1) % NM
        wait_m(t1)
        offset_src(t1)
        pltpu.async_copy(hw.at[six[t1]], rows[(t + 1) % NB], sem_g[(t + 1) % NB])

    for t in range(NM):
        meta(t, t)
    wait_m(0)
    offset_src(0)
    pltpu.async_copy(hw.at[six[0]], rows[0], sem_g[0])

    def outer(g, c):
        pend = None
        for t in range(NM):
            if pend is not None:
                pend.wait()
            else:
                @pl.when(g > 0)
                def _():
                    wait_s_boundary()
            next_gather(g, t)
            pend = scale_scatter(t)
            meta((g + 1) * NM + t, t)
        return c
    lax.fori_loop(0, NC // NM - 1, outer, 0)
    gl = NC // NM - 1
    pend = None
    for t in range(NM):
        if pend is not None:
            pend.wait()
        else:
            wait_s_boundary()
        if t < NM - 1:
            next_gather(gl, t)
        pend = scale_scatter(t)
    pend.wait()
    plsc.subcore_barrier()

    @pl.when(s < NT - 1)
    def _():
        pltpu.sync_copy(acc.at[pl.ds(row0, ROWS_PT)], agg.at[r, pl.ds(row0, ROWS_PT)])
    @pl.when(s == NT - 1)
    def _():
        tail = N - (NT - 1) * ROWS_PT
        base = pl.multiple_of((NT - 1) * ROWS_PT, 8)
        pltpu.sync_copy(acc.at[pl.ds(base, tail)], agg.at[r, pl.ds(base, tail)])


def _dot_t(a, w):
    return lax.dot_general(a, w, (((1,), (1,)), ((), ())),
                           preferred_element_type=jnp.float32)


def _head_matrix():
    lane = lax.broadcasted_iota(jnp.int32, (D, H), 0)
    hh = lax.broadcasted_iota(jnp.int32, (D, H), 1)
    return (lane // (D // H) == hh).astype(jnp.float32)


def _tc_pre_body(x_ref, g_ref, b_ref, w0_ref, hw_ref):
    x = x_ref[...]
    s1 = jnp.sum(x, axis=0, keepdims=True)
    s2 = jnp.sum(x * x, axis=0, keepdims=True)
    mean = s1 / N
    var = s2 / N - mean * mean
    xn = (x - mean) * lax.rsqrt(var + 1e-5) * g_ref[...] + b_ref[...]
    hw0 = _dot_t(xn, w0_ref[...])
    hw_ref[0] = hw0
    hw_ref[1] = hw0


def _attn_core(agg_ref, gb_ref, qkvw_ref, qkvb_ref, ow_ref, ob_ref):
    gb = gb_ref[...]
    z0 = agg_ref[0] + gb
    z1 = agg_ref[1] + gb
    qkvw = qkvw_ref[...]
    qkvb = qkvb_ref[...]
    wq, wk, wv = qkvw[:D], qkvw[D:2 * D], qkvw[2 * D:]
    bq, bk, bv = qkvb[:, :D], qkvb[:, D:2 * D], qkvb[:, 2 * D:]
    q0 = _dot_t(z0, wq) + bq
    q1 = _dot_t(z1, wq) + bq
    k0 = _dot_t(z0, wk) + bk
    k1 = _dot_t(z1, wk) + bk
    v0 = _dot_t(z0, wv) + bv
    v1 = _dot_t(z1, wv) + bv

    M = _head_matrix()
    scale = 1.0 / jnp.sqrt(jnp.float32(D // H))
    def hsum(t):
        return lax.dot_general(t, M, (((1,), (0,)), ((), ())),
                               preferred_element_type=jnp.float32)
    s00 = hsum(q0 * k0) * scale
    s01 = hsum(q0 * k1) * scale
    s10 = hsum(q1 * k0) * scale
    s11 = hsum(q1 * k1) * scale

    m0 = jnp.maximum(s00, s01)
    e00 = jnp.exp(s00 - m0)
    e01 = jnp.exp(s01 - m0)
    a00 = e00 / (e00 + e01)
    a01 = e01 / (e00 + e01)
    m1 = jnp.maximum(s10, s11)
    e10 = jnp.exp(s10 - m1)
    e11 = jnp.exp(s11 - m1)
    a10 = e10 / (e10 + e11)
    a11 = e11 / (e10 + e11)

    def hexp(a):
        return lax.dot_general(a, M, (((1,), (1,)), ((), ())),
                               preferred_element_type=jnp.float32)
    o0 = hexp(a00) * v0 + hexp(a01) * v1
    o1 = hexp(a10) * v0 + hexp(a11) * v1
    ob = ob_ref[...]
    r0 = jnp.maximum(_dot_t(o0, ow_ref[...]) + ob, 0.0)
    r1 = jnp.maximum(_dot_t(o1, ow_ref[...]) + ob, 0.0)
    return r0, r1


def _tc_attn_body(agg_ref, gb_ref, qkvw_ref, qkvb_ref, ow_ref, ob_ref, wn_ref, out_ref):
    r0, r1 = _attn_core(agg_ref, gb_ref, qkvw_ref, qkvb_ref, ow_ref, ob_ref)
    wn = wn_ref[...]
    out_ref[0] = _dot_t(r0, wn)
    out_ref[1] = _dot_t(r1, wn)


def _tc_final_body(agg_ref, gb_ref, qkvw_ref, qkvb_ref, ow_ref, ob_ref,
                   p1w_ref, p1b_ref, p2w_ref, p2b_ref, out_ref):
    r0, r1 = _attn_core(agg_ref, gb_ref, qkvw_ref, qkvb_ref, ow_ref, ob_ref)
    p1w, p1b = p1w_ref[...], p1b_ref[...]
    p2w, p2b = p2w_ref[...], p2b_ref[...]
    def mlp(t):
        h = _dot_t(t, p1w) + p1b
        h = jnp.where(h > 0, h, 0.01 * h)
        h = _dot_t(h, p2w) + p2b
        return jnp.where(h > 0, h, 0.01 * h)
    out_ref[0] = mlp(r0)
    out_ref[1] = mlp(r1)


_BN = 1000


def _full(shape):
    return pl.BlockSpec(shape, lambda i: tuple(0 for _ in shape))


def _tc_pre(x, g, b, w0):
    return pl.pallas_call(
        _tc_pre_body,
        out_shape=jax.ShapeDtypeStruct((R, N, D), jnp.float32),
    )(x, g, b, w0)


def _tc_attn(agg, gb, qkvw, qkvb, ow, ob, wn):
    grid = (N // _BN,)
    return pl.pallas_call(
        _tc_attn_body,
        grid=grid,
        in_specs=[
            pl.BlockSpec((R, _BN, D), lambda i: (0, i, 0)),
            _full((1, D)), _full((3 * D, D)), _full((1, 3 * D)),
            _full((D, D)), _full((1, D)), _full((D, D)),
        ],
        out_specs=pl.BlockSpec((R, _BN, D), lambda i: (0, i, 0)),
        out_shape=jax.ShapeDtypeStruct((R, N, D), jnp.float32),
    )(agg, gb, qkvw, qkvb, ow, ob, wn)


def _tc_final(agg, gb, qkvw, qkvb, ow, ob, p1w, p1b, p2w, p2b):
    grid = (N // _BN,)
    return pl.pallas_call(
        _tc_final_body,
        grid=grid,
        in_specs=[
            pl.BlockSpec((R, _BN, D), lambda i: (0, i, 0)),
            _full((1, D)), _full((3 * D, D)), _full((1, 3 * D)),
            _full((D, D)), _full((1, D)),
            _full((H1, D)), _full((1, H1)), _full((H2, H1)), _full((1, H2)),
        ],
        out_specs=pl.BlockSpec((R, _BN, H2), lambda i: (0, i, 0)),
        out_shape=jax.ShapeDtypeStruct((R, N, H2), jnp.float32),
    )(agg, gb, qkvw, qkvb, ow, ob, p1w, p1b, p2w, p2b)


def kernel(x, edges_weight, bn_gamma, bn_beta, gcn_W, gcn_b, qkv_w, qkv_b,
           out_w, out_b, p1_w, p1_b, p2_w, p2_b, edges_index):
    g = bn_gamma.reshape(1, D)
    b = bn_beta.reshape(1, D)
    hw = _tc_pre(x, g, b, gcn_W[0])
    edges_flat = edges_index.reshape(R * 2 * E)
    ew_flat = edges_weight.reshape(R * E)
    norm = _sc_prep(edges_flat, ew_flat)
    out = None
    for i in range(NL):
        agg = _sc_spmm(hw.reshape(R * N, D), edges_flat, norm)
        gb = gcn_b[i].reshape(1, D)
        qb = qkv_b[i].reshape(1, 3 * D)
        ob = out_b[i].reshape(1, D)
        if i < NL - 1:
            hw = _tc_attn(agg, gb, qkv_w[i], qb, out_w[i], ob, gcn_W[i + 1])
        else:
            out = _tc_final(agg, gb, qkv_w[i], qb, out_w[i], ob,
                            p1_w, p1_b.reshape(1, H1), p2_w, p2_b.reshape(1, H2))
    return out
